# baseline (speedup 1.0000x reference)
import functools

import jax
import jax.numpy as jnp
from jax import lax
from jax.experimental import pallas as pl
from jax.experimental.pallas import tpu as pltpu
from jax.experimental.pallas import tpu_sc as plsc

N = 10000
E = 320000
D = 128
NCHUNK = 4
CW = 32
NC = 2
NS = 16
B = 64
HB = B // 2
EPC = E // NC
BPC = EPC // B
NBF = BPC // NS
NXT = BPC - NBF * NS
NU = NBF // 6
IPAD = 256
ROWS_PT = 624
ROWS_LAST = N - (NS - 1) * ROWS_PT


def _mlp_body(x_ref, v_ref, w1_ref, b1_ref, w2_ref, b2_ref,
              gcat_ref, base_ref):
    x = x_ref[...]
    h = jnp.dot(x, w1_ref[...], preferred_element_type=jnp.float32) + b1_ref[...]
    h = h * jax.nn.sigmoid(h)
    y = jnp.dot(h, w2_ref[...], preferred_element_type=jnp.float32) + b2_ref[...]
    for k in range(NCHUNK):
        gcat_ref[k, :, 0:32] = y[:, 32 * k:32 * k + 32]
        gcat_ref[k, :, 32:64] = y[:, 128 + 32 * k:128 + 32 * k + 32]
        t3 = y[:, 256 + 32 * k:256 + 32 * k + 32]
        base_ref[k, :, 0:32] = x[:, 32 * k:32 * k + 32]
        for a in range(3):
            sl = v_ref[:, a, 32 * k:32 * k + 32]
            gcat_ref[k, :, 64 + 32 * a:96 + 32 * a] = t3 * sl
            base_ref[k, :, 32 + 32 * a:64 + 32 * a] = sl


def _mlp_call(x2d, vec, W1, b1, W2, b2):
    R = 1000
    grid = N // R
    return pl.pallas_call(
        _mlp_body,
        grid=(grid,),
        in_specs=[
            pl.BlockSpec((R, D), lambda i: (i, 0)),
            pl.BlockSpec((R, 3, D), lambda i: (i, 0, 0)),
            pl.BlockSpec((D, D), lambda i: (0, 0)),
            pl.BlockSpec((D,), lambda i: (0,)),
            pl.BlockSpec((D, 3 * D), lambda i: (0, 0)),
            pl.BlockSpec((3 * D,), lambda i: (0,)),
        ],
        out_specs=[
            pl.BlockSpec((NCHUNK, R, 160), lambda i: (0, i, 0)),
            pl.BlockSpec((NCHUNK, R, D), lambda i: (0, i, 0)),
        ],
        out_shape=[
            jax.ShapeDtypeStruct((NCHUNK, N, 160), jnp.float32),
            jax.ShapeDtypeStruct((NCHUNK, N, D), jnp.float32),
        ],
    )(x2d, vec, W1, b1, W2, b2)


def _sc_body(gflat, wshuf, dirpk, idxi_hbm, idxj_hbm, base_hbm,
             outacc,
             ii0, ii1, ii2, ij0, ij1, ij2, is0, is1,
             wg0, wg1, gg0, gg1, dv0, dv1, ov0, ov1, acc,
             si0, si1, si2, sg0, sg1, sw0, sw1, sd0, sd1,
             ss0, ss1):
    II = (ii0, ii1, ii2)
    IJ = (ij0, ij1, ij2)
    ISC = (is0, is1)
    WG = (wg0, wg1)
    GG = (gg0, gg1)
    DV = (dv0, dv1)
    OV = (ov0, ov1)
    SI = (si0, si1, si2)
    SG = (sg0, sg1)
    SW = (sw0, sw1)
    SD = (sd0, sd1)
    SS = (ss0, ss1)

    cid = lax.axis_index("c")
    sid = lax.axis_index("s")
    row0 = cid * BPC + sid * NBF + jnp.minimum(sid, NXT)
    rlo = sid * ROWS_PT

    def issue_idx(b, s):
        e0 = (row0 + b) * B
        pltpu.async_copy(idxi_hbm.at[pl.ds(e0, B)], II[s], SI[s])
        pltpu.async_copy(idxj_hbm.at[pl.ds(e0, B)], IJ[s], SI[s])

    def wait_idx(s):
        pltpu.make_async_copy(idxi_hbm.at[pl.ds(0, B)], II[s], SI[s]).wait()
        pltpu.make_async_copy(idxj_hbm.at[pl.ds(0, B)], IJ[s], SI[s]).wait()

    def add_koff(s, koff):
        for m in range(B // 16):
            IJ[s][pl.ds(16 * m, 16)] = IJ[s][pl.ds(16 * m, 16)] + koff

    def issue_data(b, s, s_ia, k):
        eC = jnp.minimum((row0 + b) * B, E - B)
        rC = jnp.minimum((row0 + b) * HB, E // 2 - HB)
        pltpu.async_copy(gflat.at[IJ[s_ia]], GG[s], SG[s])
        pltpu.async_copy(wshuf.at[pl.ds(eC, B), k, :], WG[s], SW[s])
        pltpu.async_copy(dirpk.at[pl.ds(rC, HB), :], DV[s], SD[s])

    def wait_data(s):
        pltpu.make_async_copy(gflat.at[IJ[0]], GG[s], SG[s]).wait()
        pltpu.make_async_copy(wshuf.at[pl.ds(0, B), 0, :], WG[s], SW[s]).wait()
        pltpu.make_async_copy(dirpk.at[pl.ds(0, HB), :], DV[s], SD[s]).wait()

    def prime_scatter(k, s):
        pltpu.async_copy(base_hbm.at[k, pl.ds(0, B), :], OV[s], SS[s])

    def wait_scatter(s):
        pltpu.make_async_copy(OV[s], acc.at[ISC[s]], SS[s]).wait()

    def compute_block(s_d, s_i):
        for m in range(B // 16):
            ISC[s_d][pl.ds(16 * m, 16)] = II[s_i][pl.ds(16 * m, 16)]
        gg, wg, dv, ov = GG[s_d], WG[s_d], DV[s_d], OV[s_d]

        @pl.loop(0, HB, unroll=2)
        def _(p):
            dvec = dv[p, pl.ds(0, 16)]
            for q in range(2):
                e = 2 * p + q
                d0 = dvec[3 * q]
                d1 = dvec[3 * q + 1]
                d2 = dvec[3 * q + 2]
                wt = [wg[e, pl.ds(16 * m, 16)] * gg[e, pl.ds(16 * m, 16)]
                      for m in range(4)]
                ov[e, pl.ds(0, 16)] = wt[0]
                ov[e, pl.ds(16, 16)] = wt[1]
                w3 = [wg[e, pl.ds(64 + 16 * mm, 16)] for mm in range(2)]
                for a, d in ((0, d0), (1, d1), (2, d2)):
                    for mm in range(2):
                        ov[e, pl.ds(32 + 32 * a + 16 * mm, 16)] = (
                            wt[2 + mm] * d
                            + w3[mm] * gg[e, pl.ds(64 + 32 * a + 16 * mm, 16)])

    def scatter(s_d):
        pltpu.async_copy(OV[s_d], acc.at[ISC[s_d]], SS[s_d], add=True)

    @pl.loop(0, NCHUNK)
    def _(k):
        koff = k * N

        @pl.when(sid < NS - 1)
        def _():
            pltpu.sync_copy(base_hbm.at[k, pl.ds(rlo, ROWS_PT), :],
                            acc.at[pl.ds(rlo, ROWS_PT), :])

        @pl.when(sid == NS - 1)
        def _():
            pltpu.sync_copy(base_hbm.at[k, pl.ds(rlo, ROWS_LAST), :],
                            acc.at[pl.ds(rlo, ROWS_LAST), :])

        plsc.subcore_barrier()

        prime_scatter(k, 0)
        prime_scatter(k, 1)
        issue_idx(0, 0)
        wait_idx(0)
        add_koff(0, koff)
        issue_data(0, 0, 0, k)
        issue_idx(1, 1)
        issue_idx(2, 2)

        @pl.loop(0, NU)
        def _(u):
            b0 = u * 6
            for j in range(6):
                b = b0 + j
                s_d, s_i = j % 2, j % 3
                s_d1, s_i1 = (j + 1) % 2, (j + 1) % 3
                wait_idx(s_i1)
                add_koff(s_i1, koff)
                issue_data(b + 1, s_d1, s_i1, k)
                wait_data(s_d)
                wait_scatter(s_d)
                compute_block(s_d, s_i)
                scatter(s_d)
                issue_idx(b + 3, s_i)

        wait_data(0)
        wait_scatter(0)
        compute_block(0, 0)

        @pl.when(sid < NXT)
        def _():
            scatter(0)
            wait_scatter(0)

        wait_scatter(1)
        wait_idx(1)
        wait_idx(2)

        plsc.subcore_barrier()

        @pl.when(sid < NS - 1)
        def _():
            pltpu.sync_copy(acc.at[pl.ds(rlo, ROWS_PT), :],
                            outacc.at[cid, k, pl.ds(rlo, ROWS_PT), :])

        @pl.when(sid == NS - 1)
        def _():
            pltpu.sync_copy(acc.at[pl.ds(rlo, ROWS_LAST), :],
                            outacc.at[cid, k, pl.ds(rlo, ROWS_LAST), :])

        plsc.subcore_barrier()


@functools.partial(jax.jit, static_argnames=())
def _sc_call(gflat, wshuf, dirpk, idx_i, idx_j, basearr):
    mesh = plsc.VectorSubcoreMesh(core_axis_name="c", subcore_axis_name="s")
    f = pl.kernel(
        _sc_body,
        out_type=jax.ShapeDtypeStruct((NC, NCHUNK, N, D), jnp.float32),
        mesh=mesh,
        scratch_types=[
            pltpu.VMEM((B,), jnp.int32),
            pltpu.VMEM((B,), jnp.int32),
            pltpu.VMEM((B,), jnp.int32),
            pltpu.VMEM((B,), jnp.int32),
            pltpu.VMEM((B,), jnp.int32),
            pltpu.VMEM((B,), jnp.int32),
            pltpu.VMEM((B,), jnp.int32),
            pltpu.VMEM((B,), jnp.int32),
            pltpu.VMEM((B, 96), jnp.float32),
            pltpu.VMEM((B, 96), jnp.float32),
            pltpu.VMEM((B, 160), jnp.float32),
            pltpu.VMEM((B, 160), jnp.float32),
            pltpu.VMEM((HB, 16), jnp.float32),
            pltpu.VMEM((HB, 16), jnp.float32),
            pltpu.VMEM((B, D), jnp.float32),
            pltpu.VMEM((B, D), jnp.float32),
            pltpu.VMEM_SHARED((N, D), jnp.float32),
            pltpu.SemaphoreType.DMA,
            pltpu.SemaphoreType.DMA,
            pltpu.SemaphoreType.DMA,
            pltpu.SemaphoreType.DMA,
            pltpu.SemaphoreType.DMA,
            pltpu.SemaphoreType.DMA,
            pltpu.SemaphoreType.DMA,
            pltpu.SemaphoreType.DMA,
            pltpu.SemaphoreType.DMA,
            pltpu.SemaphoreType.DMA,
            pltpu.SemaphoreType.DMA,
        ],
        compiler_params=pltpu.CompilerParams(use_tc_tiling_on_sc=False),
    )
    return f(gflat, wshuf, dirpk, idx_i, idx_j, basearr)


def kernel(per_atom_scalar_representation, per_atom_vector_representation,
           W_ij, dir_ij, pairlist, W1, b1, W2, b2):
    x2d = per_atom_scalar_representation.reshape(N, D)
    vec = per_atom_vector_representation
    gcat, basearr = _mlp_call(x2d, vec, W1, b1, W2, b2)
    gflat = gcat.reshape(NCHUNK * N, 160)
    wshuf = jnp.transpose(W_ij.reshape(E, 3, NCHUNK, CW),
                          (0, 2, 1, 3)).reshape(E, NCHUNK, 96)
    idx_i = jnp.pad(pairlist[0].astype(jnp.int32), (0, IPAD))
    idx_j = jnp.pad(pairlist[1].astype(jnp.int32), (0, IPAD))
    dirpk = jnp.pad(dir_ij.reshape(E // 2, 6), ((0, 0), (0, 10)))
    outacc = _sc_call(gflat, wshuf, dirpk, idx_i, idx_j, basearr)
    oa = outacc[0] + outacc[1] - basearr
    q = jnp.transpose(oa[:, :, :32], (1, 0, 2)).reshape(N, D)[:, None, :]
    mu = jnp.transpose(oa[:, :, 32:].reshape(NCHUNK, N, 3, 32),
                       (1, 2, 0, 3)).reshape(N, 3, D)
    return (q, mu)


# speedup vs baseline: 1.1869x; 1.1869x over previous
import functools

import jax
import jax.numpy as jnp
from jax import lax
from jax.experimental import pallas as pl
from jax.experimental.pallas import tpu as pltpu
from jax.experimental.pallas import tpu_sc as plsc

N = 10000
E = 320000
D = 128
NCHUNK = 4
CW = 32
NC = 2
NS = 16
B = 64
HB = B // 2
EPC = E // NC
BPC = EPC // B
NBF = BPC // NS
NXT = BPC - NBF * NS
NU = NBF // 6
IPAD = 256
ROWS_PT = 624
ROWS_LAST = N - (NS - 1) * ROWS_PT


def _mlp_body(x_ref, v_ref, w1_ref, b1_ref, w2_ref, b2_ref,
              gcat_ref, base_ref):
    x = x_ref[...]
    h = jnp.dot(x, w1_ref[...], preferred_element_type=jnp.float32) + b1_ref[...]
    h = h * jax.nn.sigmoid(h)
    y = jnp.dot(h, w2_ref[...], preferred_element_type=jnp.float32) + b2_ref[...]
    for k in range(NCHUNK):
        gcat_ref[k, :, 0:32] = y[:, 32 * k:32 * k + 32]
        gcat_ref[k, :, 32:64] = y[:, 128 + 32 * k:128 + 32 * k + 32]
        t3 = y[:, 256 + 32 * k:256 + 32 * k + 32]
        base_ref[k, :, 0:32] = x[:, 32 * k:32 * k + 32]
        for a in range(3):
            sl = v_ref[:, a, 32 * k:32 * k + 32]
            gcat_ref[k, :, 64 + 32 * a:96 + 32 * a] = t3 * sl
            base_ref[k, :, 32 + 32 * a:64 + 32 * a] = sl


def _mlp_call(x2d, vec, W1, b1, W2, b2):
    R = 1000
    grid = N // R
    return pl.pallas_call(
        _mlp_body,
        grid=(grid,),
        in_specs=[
            pl.BlockSpec((R, D), lambda i: (i, 0)),
            pl.BlockSpec((R, 3, D), lambda i: (i, 0, 0)),
            pl.BlockSpec((D, D), lambda i: (0, 0)),
            pl.BlockSpec((D,), lambda i: (0,)),
            pl.BlockSpec((D, 3 * D), lambda i: (0, 0)),
            pl.BlockSpec((3 * D,), lambda i: (0,)),
        ],
        out_specs=[
            pl.BlockSpec((NCHUNK, R, 160), lambda i: (0, i, 0)),
            pl.BlockSpec((NCHUNK, R, D), lambda i: (0, i, 0)),
        ],
        out_shape=[
            jax.ShapeDtypeStruct((NCHUNK, N, 160), jnp.float32),
            jax.ShapeDtypeStruct((NCHUNK, N, D), jnp.float32),
        ],
    )(x2d, vec, W1, b1, W2, b2)


def _sc_body(gflat, wij, dirpk, idxi_hbm, idxj_hbm, base_hbm,
             outacc,
             ii0, ii1, ii2, ij0, ij1, ij2, is0, is1,
             wg0, wg1, gg0, gg1, dv0, dv1, ov0, ov1, acc,
             si0, si1, si2, sg0, sg1, sw0, sw1, sd0, sd1,
             ss0, ss1):
    II = (ii0, ii1, ii2)
    IJ = (ij0, ij1, ij2)
    ISC = (is0, is1)
    WG = (wg0, wg1)
    GG = (gg0, gg1)
    DV = (dv0, dv1)
    OV = (ov0, ov1)
    SI = (si0, si1, si2)
    SG = (sg0, sg1)
    SW = (sw0, sw1)
    SD = (sd0, sd1)
    SS = (ss0, ss1)

    cid = lax.axis_index("c")
    sid = lax.axis_index("s")
    row0 = cid * BPC + sid * NBF + jnp.minimum(sid, NXT)
    rlo = sid * ROWS_PT

    def issue_idx(b, s):
        e0 = (row0 + b) * B
        pltpu.async_copy(idxi_hbm.at[pl.ds(e0, B)], II[s], SI[s])
        pltpu.async_copy(idxj_hbm.at[pl.ds(e0, B)], IJ[s], SI[s])

    def wait_idx(s):
        pltpu.make_async_copy(idxi_hbm.at[pl.ds(0, B)], II[s], SI[s]).wait()
        pltpu.make_async_copy(idxj_hbm.at[pl.ds(0, B)], IJ[s], SI[s]).wait()

    def add_koff(s, koff):
        for m in range(B // 16):
            IJ[s][pl.ds(16 * m, 16)] = IJ[s][pl.ds(16 * m, 16)] + koff

    def issue_data(b, s, s_ia, k):
        eC = jnp.minimum((row0 + b) * B, E - B)
        rC = jnp.minimum((row0 + b) * HB, E // 2 - HB)
        pltpu.async_copy(gflat.at[IJ[s_ia]], GG[s], SG[s])
        for p in range(3):
            pltpu.async_copy(wij.at[pl.ds(eC, B), pl.ds(128 * p + CW * k, CW)],
                             WG[s].at[:, pl.ds(32 * p, 32)], SW[s])
        pltpu.async_copy(dirpk.at[pl.ds(rC, HB), :], DV[s], SD[s])

    def wait_data(s):
        pltpu.make_async_copy(gflat.at[IJ[0]], GG[s], SG[s]).wait()
        for p in range(3):
            pltpu.make_async_copy(wij.at[pl.ds(0, B), pl.ds(128 * p, CW)],
                                  WG[s].at[:, pl.ds(32 * p, 32)], SW[s]).wait()
        pltpu.make_async_copy(dirpk.at[pl.ds(0, HB), :], DV[s], SD[s]).wait()

    def prime_scatter(k, s):
        pltpu.async_copy(base_hbm.at[k, pl.ds(0, B), :], OV[s], SS[s])

    def wait_scatter(s):
        pltpu.make_async_copy(OV[s], acc.at[ISC[s]], SS[s]).wait()

    def compute_block(s_d, s_i):
        for m in range(B // 16):
            ISC[s_d][pl.ds(16 * m, 16)] = II[s_i][pl.ds(16 * m, 16)]
        gg, wg, dv, ov = GG[s_d], WG[s_d], DV[s_d], OV[s_d]

        @pl.loop(0, HB, unroll=2)
        def _(p):
            dvec = dv[p, pl.ds(0, 16)]
            for q in range(2):
                e = 2 * p + q
                d0 = dvec[3 * q]
                d1 = dvec[3 * q + 1]
                d2 = dvec[3 * q + 2]
                wt = [wg[e, pl.ds(16 * m, 16)] * gg[e, pl.ds(16 * m, 16)]
                      for m in range(4)]
                ov[e, pl.ds(0, 16)] = wt[0]
                ov[e, pl.ds(16, 16)] = wt[1]
                w3 = [wg[e, pl.ds(64 + 16 * mm, 16)] for mm in range(2)]
                for a, d in ((0, d0), (1, d1), (2, d2)):
                    for mm in range(2):
                        ov[e, pl.ds(32 + 32 * a + 16 * mm, 16)] = (
                            wt[2 + mm] * d
                            + w3[mm] * gg[e, pl.ds(64 + 32 * a + 16 * mm, 16)])

    def scatter(s_d):
        pltpu.async_copy(OV[s_d], acc.at[ISC[s_d]], SS[s_d], add=True)

    @pl.loop(0, NCHUNK)
    def _(k):
        koff = k * N

        @pl.when(sid < NS - 1)
        def _():
            pltpu.sync_copy(base_hbm.at[k, pl.ds(rlo, ROWS_PT), :],
                            acc.at[pl.ds(rlo, ROWS_PT), :])

        @pl.when(sid == NS - 1)
        def _():
            pltpu.sync_copy(base_hbm.at[k, pl.ds(rlo, ROWS_LAST), :],
                            acc.at[pl.ds(rlo, ROWS_LAST), :])

        plsc.subcore_barrier()

        prime_scatter(k, 0)
        prime_scatter(k, 1)
        issue_idx(0, 0)
        wait_idx(0)
        add_koff(0, koff)
        issue_data(0, 0, 0, k)
        issue_idx(1, 1)
        issue_idx(2, 2)

        @pl.loop(0, NU)
        def _(u):
            b0 = u * 6
            for j in range(6):
                b = b0 + j
                s_d, s_i = j % 2, j % 3
                s_d1, s_i1 = (j + 1) % 2, (j + 1) % 3
                wait_idx(s_i1)
                add_koff(s_i1, koff)
                issue_data(b + 1, s_d1, s_i1, k)
                wait_data(s_d)
                wait_scatter(s_d)
                compute_block(s_d, s_i)
                scatter(s_d)
                issue_idx(b + 3, s_i)

        wait_data(0)
        wait_scatter(0)
        compute_block(0, 0)

        @pl.when(sid < NXT)
        def _():
            scatter(0)
            wait_scatter(0)

        wait_scatter(1)
        wait_idx(1)
        wait_idx(2)

        plsc.subcore_barrier()

        @pl.when(sid < NS - 1)
        def _():
            pltpu.sync_copy(acc.at[pl.ds(rlo, ROWS_PT), :],
                            outacc.at[cid, k, pl.ds(rlo, ROWS_PT), :])

        @pl.when(sid == NS - 1)
        def _():
            pltpu.sync_copy(acc.at[pl.ds(rlo, ROWS_LAST), :],
                            outacc.at[cid, k, pl.ds(rlo, ROWS_LAST), :])

        plsc.subcore_barrier()


@functools.partial(jax.jit, static_argnames=())
def _sc_call(gflat, W_ij, dirpk, idx_i, idx_j, basearr):
    mesh = plsc.VectorSubcoreMesh(core_axis_name="c", subcore_axis_name="s")
    f = pl.kernel(
        _sc_body,
        out_type=jax.ShapeDtypeStruct((NC, NCHUNK, N, D), jnp.float32),
        mesh=mesh,
        scratch_types=[
            pltpu.VMEM((B,), jnp.int32),
            pltpu.VMEM((B,), jnp.int32),
            pltpu.VMEM((B,), jnp.int32),
            pltpu.VMEM((B,), jnp.int32),
            pltpu.VMEM((B,), jnp.int32),
            pltpu.VMEM((B,), jnp.int32),
            pltpu.VMEM((B,), jnp.int32),
            pltpu.VMEM((B,), jnp.int32),
            pltpu.VMEM((B, 96), jnp.float32),
            pltpu.VMEM((B, 96), jnp.float32),
            pltpu.VMEM((B, 160), jnp.float32),
            pltpu.VMEM((B, 160), jnp.float32),
            pltpu.VMEM((HB, 16), jnp.float32),
            pltpu.VMEM((HB, 16), jnp.float32),
            pltpu.VMEM((B, D), jnp.float32),
            pltpu.VMEM((B, D), jnp.float32),
            pltpu.VMEM_SHARED((N, D), jnp.float32),
            pltpu.SemaphoreType.DMA,
            pltpu.SemaphoreType.DMA,
            pltpu.SemaphoreType.DMA,
            pltpu.SemaphoreType.DMA,
            pltpu.SemaphoreType.DMA,
            pltpu.SemaphoreType.DMA,
            pltpu.SemaphoreType.DMA,
            pltpu.SemaphoreType.DMA,
            pltpu.SemaphoreType.DMA,
            pltpu.SemaphoreType.DMA,
            pltpu.SemaphoreType.DMA,
        ],
        compiler_params=pltpu.CompilerParams(use_tc_tiling_on_sc=False),
    )
    return f(gflat, W_ij, dirpk, idx_i, idx_j, basearr)


def kernel(per_atom_scalar_representation, per_atom_vector_representation,
           W_ij, dir_ij, pairlist, W1, b1, W2, b2):
    x2d = per_atom_scalar_representation.reshape(N, D)
    vec = per_atom_vector_representation
    gcat, basearr = _mlp_call(x2d, vec, W1, b1, W2, b2)
    gflat = gcat.reshape(NCHUNK * N, 160)
    idx_i = jnp.pad(pairlist[0].astype(jnp.int32), (0, IPAD))
    idx_j = jnp.pad(pairlist[1].astype(jnp.int32), (0, IPAD))
    dirpk = jnp.pad(dir_ij.reshape(E // 2, 6), ((0, 0), (0, 10)))
    outacc = _sc_call(gflat, W_ij, dirpk, idx_i, idx_j, basearr)
    oa = outacc[0] + outacc[1] - basearr
    q = jnp.transpose(oa[:, :, :32], (1, 0, 2)).reshape(N, D)[:, None, :]
    mu = jnp.transpose(oa[:, :, 32:].reshape(NCHUNK, N, 3, 32),
                       (1, 2, 0, 3)).reshape(N, 3, D)
    return (q, mu)


# speedup vs baseline: 1.4385x; 1.2120x over previous
import functools

import jax
import jax.numpy as jnp
from jax import lax
from jax.experimental import pallas as pl
from jax.experimental.pallas import tpu as pltpu
from jax.experimental.pallas import tpu_sc as plsc

N = 10000
E = 320000
D = 128
NCHUNK = 4
CW = 32
NC = 2
NS = 16
B = 64
HB = B // 2
EPC = E // NC
BPC = EPC // B
NBF = BPC // NS
NXT = BPC - NBF * NS
NU = NBF // 6
IPAD = 256
ROWS_PT = 624
ROWS_LAST = N - (NS - 1) * ROWS_PT


def _mlp_body(x_ref, v_ref, w1_ref, b1_ref, w2_ref, b2_ref,
              gcat_ref, base_ref):
    x = x_ref[...]
    h = jnp.dot(x, w1_ref[...], preferred_element_type=jnp.float32) + b1_ref[...]
    h = h * jax.nn.sigmoid(h)
    y = jnp.dot(h, w2_ref[...], preferred_element_type=jnp.float32) + b2_ref[...]
    for k in range(NCHUNK):
        gcat_ref[k, :, 0:32] = y[:, 32 * k:32 * k + 32]
        gcat_ref[k, :, 32:64] = y[:, 128 + 32 * k:128 + 32 * k + 32]
        t3 = y[:, 256 + 32 * k:256 + 32 * k + 32]
        base_ref[k, :, 0:32] = x[:, 32 * k:32 * k + 32]
        for a in range(3):
            sl = v_ref[:, a, 32 * k:32 * k + 32]
            gcat_ref[k, :, 64 + 32 * a:96 + 32 * a] = t3 * sl
            base_ref[k, :, 32 + 32 * a:64 + 32 * a] = sl


def _mlp_call(x2d, vec, W1, b1, W2, b2):
    R = 1000
    grid = N // R
    return pl.pallas_call(
        _mlp_body,
        grid=(grid,),
        in_specs=[
            pl.BlockSpec((R, D), lambda i: (i, 0)),
            pl.BlockSpec((R, 3, D), lambda i: (i, 0, 0)),
            pl.BlockSpec((D, D), lambda i: (0, 0)),
            pl.BlockSpec((D,), lambda i: (0,)),
            pl.BlockSpec((D, 3 * D), lambda i: (0, 0)),
            pl.BlockSpec((3 * D,), lambda i: (0,)),
        ],
        out_specs=[
            pl.BlockSpec((NCHUNK, R, 160), lambda i: (0, i, 0)),
            pl.BlockSpec((NCHUNK, R, D), lambda i: (0, i, 0)),
        ],
        out_shape=[
            jax.ShapeDtypeStruct((NCHUNK, N, 160), jnp.float32),
            jax.ShapeDtypeStruct((NCHUNK, N, D), jnp.float32),
        ],
    )(x2d, vec, W1, b1, W2, b2)


def _wperm_body(w_ref, o_ref):
    w = w_ref[...]
    for k in range(NCHUNK):
        o_ref[k, :, 0:32] = w[:, 32 * k:32 * k + 32]
        o_ref[k, :, 32:64] = w[:, 128 + 32 * k:128 + 32 * k + 32]
        o_ref[k, :, 64:96] = w[:, 256 + 32 * k:256 + 32 * k + 32]
        o_ref[k, :, 96:128] = jnp.zeros((w.shape[0], 32), jnp.float32)


def _wperm_call(W_ij):
    RB = 2000
    return pl.pallas_call(
        _wperm_body,
        grid=(E // RB,),
        in_specs=[pl.BlockSpec((RB, 3 * D), lambda i: (i, 0))],
        out_specs=pl.BlockSpec((NCHUNK, RB, D), lambda i: (0, i, 0)),
        out_shape=jax.ShapeDtypeStruct((NCHUNK, E, D), jnp.float32),
    )(W_ij)


def _sc_body(gflat, wcat, dirpk, idxi_hbm, idxj_hbm, base_hbm,
             outacc,
             ii0, ii1, ii2, ij0, ij1, ij2, is0, is1,
             wg0, wg1, gg0, gg1, dv0, dv1, ov0, ov1, acc,
             si0, si1, si2, sg0, sg1, sw0, sw1, sd0, sd1,
             ss0, ss1):
    II = (ii0, ii1, ii2)
    IJ = (ij0, ij1, ij2)
    ISC = (is0, is1)
    WG = (wg0, wg1)
    GG = (gg0, gg1)
    DV = (dv0, dv1)
    OV = (ov0, ov1)
    SI = (si0, si1, si2)
    SG = (sg0, sg1)
    SW = (sw0, sw1)
    SD = (sd0, sd1)
    SS = (ss0, ss1)

    cid = lax.axis_index("c")
    sid = lax.axis_index("s")
    row0 = cid * BPC + sid * NBF + jnp.minimum(sid, NXT)
    rlo = sid * ROWS_PT

    def issue_idx(b, s):
        e0 = (row0 + b) * B
        pltpu.async_copy(idxi_hbm.at[pl.ds(e0, B)], II[s], SI[s])
        pltpu.async_copy(idxj_hbm.at[pl.ds(e0, B)], IJ[s], SI[s])

    def wait_idx(s):
        pltpu.make_async_copy(idxi_hbm.at[pl.ds(0, B)], II[s], SI[s]).wait()
        pltpu.make_async_copy(idxj_hbm.at[pl.ds(0, B)], IJ[s], SI[s]).wait()

    def add_koff(s, koff):
        for m in range(B // 16):
            IJ[s][pl.ds(16 * m, 16)] = IJ[s][pl.ds(16 * m, 16)] + koff

    def issue_data(b, s, s_ia, k):
        eC = jnp.minimum((row0 + b) * B, E - B)
        rC = jnp.minimum((row0 + b) * HB, E // 2 - HB)
        pltpu.async_copy(gflat.at[IJ[s_ia]], GG[s], SG[s])
        pltpu.async_copy(wcat.at[k, pl.ds(eC, B), pl.ds(0, 96)], WG[s], SW[s])
        pltpu.async_copy(dirpk.at[pl.ds(rC, HB), :], DV[s], SD[s])

    def wait_data(s):
        pltpu.make_async_copy(gflat.at[IJ[0]], GG[s], SG[s]).wait()
        pltpu.make_async_copy(wcat.at[0, pl.ds(0, B), pl.ds(0, 96)], WG[s], SW[s]).wait()
        pltpu.make_async_copy(dirpk.at[pl.ds(0, HB), :], DV[s], SD[s]).wait()

    def prime_scatter(k, s):
        pltpu.async_copy(base_hbm.at[k, pl.ds(0, B), :], OV[s], SS[s])

    def wait_scatter(s):
        pltpu.make_async_copy(OV[s], acc.at[ISC[s]], SS[s]).wait()

    def compute_block(s_d, s_i):
        for m in range(B // 16):
            ISC[s_d][pl.ds(16 * m, 16)] = II[s_i][pl.ds(16 * m, 16)]
        gg, wg, dv, ov = GG[s_d], WG[s_d], DV[s_d], OV[s_d]

        @pl.loop(0, HB, unroll=2)
        def _(p):
            dvec = dv[p, pl.ds(0, 16)]
            for q in range(2):
                e = 2 * p + q
                d0 = dvec[3 * q]
                d1 = dvec[3 * q + 1]
                d2 = dvec[3 * q + 2]
                wt = [wg[e, pl.ds(16 * m, 16)] * gg[e, pl.ds(16 * m, 16)]
                      for m in range(4)]
                ov[e, pl.ds(0, 16)] = wt[0]
                ov[e, pl.ds(16, 16)] = wt[1]
                w3 = [wg[e, pl.ds(64 + 16 * mm, 16)] for mm in range(2)]
                for a, d in ((0, d0), (1, d1), (2, d2)):
                    for mm in range(2):
                        ov[e, pl.ds(32 + 32 * a + 16 * mm, 16)] = (
                            wt[2 + mm] * d
                            + w3[mm] * gg[e, pl.ds(64 + 32 * a + 16 * mm, 16)])

    def scatter(s_d):
        pltpu.async_copy(OV[s_d], acc.at[ISC[s_d]], SS[s_d], add=True)

    @pl.loop(0, NCHUNK)
    def _(k):
        koff = k * N

        @pl.when(sid < NS - 1)
        def _():
            pltpu.sync_copy(base_hbm.at[k, pl.ds(rlo, ROWS_PT), :],
                            acc.at[pl.ds(rlo, ROWS_PT), :])

        @pl.when(sid == NS - 1)
        def _():
            pltpu.sync_copy(base_hbm.at[k, pl.ds(rlo, ROWS_LAST), :],
                            acc.at[pl.ds(rlo, ROWS_LAST), :])

        plsc.subcore_barrier()

        prime_scatter(k, 0)
        prime_scatter(k, 1)
        issue_idx(0, 0)
        wait_idx(0)
        add_koff(0, koff)
        issue_data(0, 0, 0, k)
        issue_idx(1, 1)
        issue_idx(2, 2)

        @pl.loop(0, NU)
        def _(u):
            b0 = u * 6
            for j in range(6):
                b = b0 + j
                s_d, s_i = j % 2, j % 3
                s_d1, s_i1 = (j + 1) % 2, (j + 1) % 3
                wait_idx(s_i1)
                add_koff(s_i1, koff)
                issue_data(b + 1, s_d1, s_i1, k)
                wait_data(s_d)
                wait_scatter(s_d)
                compute_block(s_d, s_i)
                scatter(s_d)
                issue_idx(b + 3, s_i)

        wait_data(0)
        wait_scatter(0)
        compute_block(0, 0)

        @pl.when(sid < NXT)
        def _():
            scatter(0)
            wait_scatter(0)

        wait_scatter(1)
        wait_idx(1)
        wait_idx(2)

        plsc.subcore_barrier()

        @pl.when(sid < NS - 1)
        def _():
            pltpu.sync_copy(acc.at[pl.ds(rlo, ROWS_PT), :],
                            outacc.at[cid, k, pl.ds(rlo, ROWS_PT), :])

        @pl.when(sid == NS - 1)
        def _():
            pltpu.sync_copy(acc.at[pl.ds(rlo, ROWS_LAST), :],
                            outacc.at[cid, k, pl.ds(rlo, ROWS_LAST), :])

        plsc.subcore_barrier()


@functools.partial(jax.jit, static_argnames=())
def _sc_call(gflat, wcat, dirpk, idx_i, idx_j, basearr):
    mesh = plsc.VectorSubcoreMesh(core_axis_name="c", subcore_axis_name="s")
    f = pl.kernel(
        _sc_body,
        out_type=jax.ShapeDtypeStruct((NC, NCHUNK, N, D), jnp.float32),
        mesh=mesh,
        scratch_types=[
            pltpu.VMEM((B,), jnp.int32),
            pltpu.VMEM((B,), jnp.int32),
            pltpu.VMEM((B,), jnp.int32),
            pltpu.VMEM((B,), jnp.int32),
            pltpu.VMEM((B,), jnp.int32),
            pltpu.VMEM((B,), jnp.int32),
            pltpu.VMEM((B,), jnp.int32),
            pltpu.VMEM((B,), jnp.int32),
            pltpu.VMEM((B, 96), jnp.float32),
            pltpu.VMEM((B, 96), jnp.float32),
            pltpu.VMEM((B, 160), jnp.float32),
            pltpu.VMEM((B, 160), jnp.float32),
            pltpu.VMEM((HB, 16), jnp.float32),
            pltpu.VMEM((HB, 16), jnp.float32),
            pltpu.VMEM((B, D), jnp.float32),
            pltpu.VMEM((B, D), jnp.float32),
            pltpu.VMEM_SHARED((N, D), jnp.float32),
            pltpu.SemaphoreType.DMA,
            pltpu.SemaphoreType.DMA,
            pltpu.SemaphoreType.DMA,
            pltpu.SemaphoreType.DMA,
            pltpu.SemaphoreType.DMA,
            pltpu.SemaphoreType.DMA,
            pltpu.SemaphoreType.DMA,
            pltpu.SemaphoreType.DMA,
            pltpu.SemaphoreType.DMA,
            pltpu.SemaphoreType.DMA,
            pltpu.SemaphoreType.DMA,
        ],
        compiler_params=pltpu.CompilerParams(use_tc_tiling_on_sc=False),
    )
    return f(gflat, wcat, dirpk, idx_i, idx_j, basearr)


def kernel(per_atom_scalar_representation, per_atom_vector_representation,
           W_ij, dir_ij, pairlist, W1, b1, W2, b2):
    x2d = per_atom_scalar_representation.reshape(N, D)
    vec = per_atom_vector_representation
    gcat, basearr = _mlp_call(x2d, vec, W1, b1, W2, b2)
    gflat = gcat.reshape(NCHUNK * N, 160)
    wcat = _wperm_call(W_ij)
    idx_i = jnp.pad(pairlist[0].astype(jnp.int32), (0, IPAD))
    idx_j = jnp.pad(pairlist[1].astype(jnp.int32), (0, IPAD))
    dirpk = jnp.pad(dir_ij.reshape(E // 2, 6), ((0, 0), (0, 10)))
    outacc = _sc_call(gflat, wcat, dirpk, idx_i, idx_j, basearr)
    oa = outacc[0] + outacc[1] - basearr
    q = jnp.transpose(oa[:, :, :32], (1, 0, 2)).reshape(N, D)[:, None, :]
    mu = jnp.transpose(oa[:, :, 32:].reshape(NCHUNK, N, 3, 32),
                       (1, 2, 0, 3)).reshape(N, 3, D)
    return (q, mu)
